# Initial kernel scaffold; baseline (speedup 1.0000x reference)
import jax
import jax.numpy as jnp
from jax.experimental import pallas as pl


def kernel(points):
    raise NotImplementedError("write your pallas kernel here")


# speedup vs baseline: 4.4273x; 4.4273x over previous
import functools

import jax
import jax.numpy as jnp
from jax import lax
from jax.experimental import pallas as pl
from jax.experimental.pallas import tpu as pltpu
from jax.experimental.pallas import tpu_sc as plsc

GX = 100
GY = 100
SENT = GX * GY
MAXV = 16000
MAXP = 32

NTILES = 16
NBINS = 10240
BINS_PER_TILE = NBINS // NTILES
CHUNK = 12544
SUB = 6272
NPAD = NTILES * CHUNK
GROUPS = SUB // 16

ROWS = MAXV * MAXP
ROWS_TOT = 512256
ZROWS = ROWS_TOT // NTILES
NROWS_TOT = 16256
NZROWS = NROWS_TOT // NTILES

_mesh = plsc.VectorSubcoreMesh(core_axis_name="c", subcore_axis_name="s",
                               num_cores=1)


@functools.partial(
    pl.kernel,
    out_type=[
        jax.ShapeDtypeStruct((ROWS_TOT, 8), jnp.float32),
        jax.ShapeDtypeStruct((NROWS_TOT, 8), jnp.float32),
        jax.ShapeDtypeStruct((NTILES * NBINS,), jnp.int32),
        jax.ShapeDtypeStruct((NTILES * NBINS,), jnp.int32),
        jax.ShapeDtypeStruct((NTILES * 16,), jnp.int32),
        jax.ShapeDtypeStruct((NBINS,), jnp.int32),
    ],
    mesh=_mesh,
    compiler_params=pltpu.CompilerParams(
        needs_layout_passes=False, use_tc_tiling_on_sc=False),
    scratch_types=[
        pltpu.VMEM((SUB, 8), jnp.float32),
        pltpu.VMEM((CHUNK,), jnp.int32),
        pltpu.VMEM((NBINS,), jnp.int32),
        pltpu.VMEM((NBINS,), jnp.int32),
        pltpu.VMEM((BINS_PER_TILE,), jnp.int32),
        pltpu.VMEM((BINS_PER_TILE,), jnp.int32),
        pltpu.VMEM((2 * SUB // 128, 128), jnp.int32),
        pltpu.VMEM((1024, 8), jnp.float32),
        pltpu.VMEM((BINS_PER_TILE, 8), jnp.float32),
        pltpu.VMEM((BINS_PER_TILE // 128, 128), jnp.int32),
        pltpu.VMEM((256,), jnp.int32),
        pltpu.VMEM((16,), jnp.int32),
        pltpu.SemaphoreType.DMA,
        pltpu.SemaphoreType.DMA,
    ],
)
def _voxelize_sc(pts8, out8, nout, hists, offs, occ_cnts, rank_hbm,
                 ptsv, linv, ctrv, rankv, totv, rankb, idxv, zerov, nrows,
                 nidx, occv, vtmp, sem, sem2):
    tid = lax.axis_index("s")
    lane = lax.iota(jnp.int32, 16)
    zvec = jnp.zeros((16,), jnp.float32)

    def zero_fill(g, _):
        r = g * 16 + lane
        for k in range(8):
            plsc.store_scatter(zerov, [r, jnp.full((16,), k, jnp.int32)], zvec)
        return 0
    lax.fori_loop(0, 64, zero_fill, 0)

    def zero_f32(j, _):
        pltpu.sync_copy(zerov, out8.at[pl.ds(tid * ZROWS + j * 1024, 1024)])
        return 0
    lax.fori_loop(0, 31, zero_f32, 0)
    pltpu.sync_copy(zerov.at[pl.ds(0, ZROWS - 31 * 1024)],
                    out8.at[pl.ds(tid * ZROWS + 31 * 1024, ZROWS - 31 * 1024)])
    pltpu.sync_copy(zerov.at[pl.ds(0, NZROWS)],
                    nout.at[pl.ds(tid * NZROWS, NZROWS)])

    def zero_ctr(g, _):
        ctrv[pl.ds(g * 16, 16)] = jnp.zeros((16,), jnp.int32)
        return 0
    lax.fori_loop(0, NBINS // 16, zero_ctr, 0)

    for sub in range(2):
        pltpu.sync_copy(pts8.at[pl.ds((tid * 2 + sub) * SUB, SUB)], ptsv)

        def bin_group(g, _):
            r = g * 16 + lane
            px = plsc.load_gather(ptsv, [r, jnp.full((16,), 0, jnp.int32)])
            py = plsc.load_gather(ptsv, [r, jnp.full((16,), 1, jnp.int32)])
            pz = plsc.load_gather(ptsv, [r, jnp.full((16,), 2, jnp.int32)])
            xi = (px / jnp.float32(0.01)).astype(jnp.int32)
            yi = (py / jnp.float32(0.01)).astype(jnp.int32)
            zi = (pz / jnp.float32(1.0)).astype(jnp.int32)
            inr = ((xi >= 0) & (xi < GX) & (yi >= 0) & (yi < GY)
                   & (zi >= 0) & (zi < 1))
            ln = jnp.where(inr, zi * SENT + yi * GX + xi, SENT)
            linv[pl.ds(sub * SUB + g * 16, 16)] = ln
            old = plsc.load_gather(ctrv, [ln])
            cnt, lastm = plsc.scan_count(ln)
            plsc.store_scatter(ctrv, [ln], old + cnt, mask=lastm)
            return 0
        lax.fori_loop(0, GROUPS, bin_group, 0)

    pltpu.sync_copy(ctrv, hists.at[pl.ds(tid * NBINS, NBINS)])
    plsc.subcore_barrier()

    b0 = tid * BINS_PER_TILE
    for c in range(NTILES):
        pltpu.sync_copy(hists.at[pl.ds(c * NBINS + b0, BINS_PER_TILE)],
                        rankv.at[pl.ds(c * BINS_PER_TILE, BINS_PER_TILE)])

    def prefix_group(g, _):
        acc = jnp.zeros((16,), jnp.int32)
        for c in range(NTILES):
            h = rankv[pl.ds(c * BINS_PER_TILE + g * 16, 16)]
            rankv[pl.ds(c * BINS_PER_TILE + g * 16, 16)] = acc
            acc = acc + h
        totv[pl.ds(g * 16, 16)] = acc
        return 0
    lax.fori_loop(0, BINS_PER_TILE // 16, prefix_group, 0)

    for c in range(NTILES):
        pltpu.sync_copy(rankv.at[pl.ds(c * BINS_PER_TILE, BINS_PER_TILE)],
                        offs.at[pl.ds(c * NBINS + b0, BINS_PER_TILE)])

    def occ_group(g, a):
        tot = totv[pl.ds(g * 16, 16)]
        binv = b0 + g * 16 + lane
        occ = (tot > 0) & (binv < SENT)
        return a + occ.astype(jnp.int32)
    occ_acc = lax.fori_loop(0, BINS_PER_TILE // 16, occ_group,
                            jnp.zeros((16,), jnp.int32))
    vtmp[...] = jnp.full((16,), jnp.sum(occ_acc), jnp.int32)
    pltpu.sync_copy(vtmp, occ_cnts.at[pl.ds(tid * 16, 16)])
    plsc.subcore_barrier()

    pltpu.sync_copy(occ_cnts, occv)
    diag = plsc.load_gather(occv, [lane * 17])
    rank_base = jnp.sum(jnp.where(lane < tid, diag, 0))

    def rank_group(g, rbase):
        tot = totv[pl.ds(g * 16, 16)]
        binv = b0 + g * 16 + lane
        occ = (tot > 0) & (binv < SENT)
        occ_i = occ.astype(jnp.int32)
        excl = plsc.cumsum(occ_i) - occ_i
        rank = rbase + excl
        rankb[pl.ds(g * 16, 16)] = rank
        r = g * 16 + lane
        plsc.store_scatter(nrows, [r, jnp.full((16,), 0, jnp.int32)],
                           jnp.minimum(tot, MAXP).astype(jnp.float32))
        plsc.store_scatter(nrows, [r, jnp.full((16,), 1, jnp.int32)],
                           (binv // GX).astype(jnp.float32))
        plsc.store_scatter(nrows, [r, jnp.full((16,), 2, jnp.int32)],
                           (binv % GX).astype(jnp.float32))
        fi = g * 16 + lane
        plsc.store_scatter(nidx, [fi // 128, fi % 128],
                           jnp.where(occ, rank, MAXV + lane))
        return rbase + jnp.sum(occ_i)
    lax.fori_loop(0, BINS_PER_TILE // 16, rank_group, rank_base)

    cps = [pltpu.async_copy(nrows.at[pl.ds(j * 128, 128)],
                            nout.at[nidx.at[j]], sem)
           for j in range(BINS_PER_TILE // 128)]
    for cp in cps:
        cp.wait()
    pltpu.sync_copy(rankb, rank_hbm.at[pl.ds(b0, BINS_PER_TILE)])
    plsc.subcore_barrier()

    pltpu.sync_copy(offs.at[pl.ds(tid * NBINS, NBINS)], ctrv)
    pltpu.sync_copy(rank_hbm, rankv)

    for sub in range(2):
        pltpu.sync_copy(pts8.at[pl.ds((tid * 2 + sub) * SUB, SUB)], ptsv)

        def slot_group(g, _):
            ln = linv[pl.ds(sub * SUB + g * 16, 16)]
            old = plsc.load_gather(ctrv, [ln])
            cnt, lastm = plsc.scan_count(ln)
            slot = old + cnt - 1
            plsc.store_scatter(ctrv, [ln], old + cnt, mask=lastm)
            row = plsc.load_gather(rankv, [ln])
            valid = (ln < SENT) & (slot < MAXP)
            dst = jnp.where(valid, row * MAXP + slot, ROWS + lane)
            fi = sub * SUB + g * 16 + lane
            plsc.store_scatter(idxv, [fi // 128, fi % 128], dst)
            return 0
        lax.fori_loop(0, GROUPS, slot_group, 0)

        def scat_block(b, _):
            for k in range(7):
                pltpu.async_copy(
                    ptsv.at[pl.ds((b * 7 + k) * 128, 128)],
                    out8.at[idxv.at[sub * (SUB // 128) + b * 7 + k]],
                    sem2).wait()
            return 0
        lax.fori_loop(0, 7, scat_block, 0)


def kernel(points):
    n = points.shape[0]
    pts = jnp.concatenate(
        [points, jnp.full((NPAD - n, 4), 2.0, jnp.float32)], axis=0)
    pts8 = jnp.concatenate(
        [pts, jnp.zeros((NPAD, 4), jnp.float32)], axis=1)
    out8, nout, _, _, _, _ = _voxelize_sc(pts8)
    vox = out8[:ROWS, :4].reshape(MAXV, MAXP, 4)
    npts = nout[:MAXV, 0].astype(jnp.int64)
    coords = jnp.stack(
        [jnp.zeros((MAXV,), jnp.int64),
         nout[:MAXV, 1].astype(jnp.int64),
         nout[:MAXV, 2].astype(jnp.int64)], axis=1)
    return vox, coords, npts


# speedup vs baseline: 4.5754x; 1.0335x over previous
import functools

import jax
import jax.numpy as jnp
from jax import lax
from jax.experimental import pallas as pl
from jax.experimental.pallas import tpu as pltpu
from jax.experimental.pallas import tpu_sc as plsc

GX = 100
GY = 100
SENT = GX * GY
MAXV = 16000
MAXP = 32

NTILES = 16
NBINS = 10240
BINS_PER_TILE = NBINS // NTILES
CHUNK = 12544
SUB = 6272
NPAD = NTILES * CHUNK
GROUPS = SUB // 16

ROWS = MAXV * MAXP
ROWS_TOT = 512256
ZROWS = ROWS_TOT // NTILES
NROWS_TOT = 16256
NZROWS = NROWS_TOT // NTILES

_mesh = plsc.VectorSubcoreMesh(core_axis_name="c", subcore_axis_name="s",
                               num_cores=1)


@functools.partial(
    pl.kernel,
    out_type=[
        jax.ShapeDtypeStruct((ROWS_TOT, 8), jnp.float32),
        jax.ShapeDtypeStruct((NROWS_TOT, 8), jnp.float32),
        jax.ShapeDtypeStruct((NTILES * NBINS,), jnp.int32),
        jax.ShapeDtypeStruct((NTILES * NBINS,), jnp.int32),
        jax.ShapeDtypeStruct((NTILES * 16,), jnp.int32),
        jax.ShapeDtypeStruct((NBINS,), jnp.int32),
    ],
    mesh=_mesh,
    compiler_params=pltpu.CompilerParams(
        needs_layout_passes=False, use_tc_tiling_on_sc=False),
    scratch_types=[
        pltpu.VMEM((SUB, 8), jnp.float32),
        pltpu.VMEM((CHUNK,), jnp.int32),
        pltpu.VMEM((NBINS,), jnp.int32),
        pltpu.VMEM((NBINS,), jnp.int32),
        pltpu.VMEM((BINS_PER_TILE,), jnp.int32),
        pltpu.VMEM((BINS_PER_TILE,), jnp.int32),
        pltpu.VMEM((2 * SUB // 128, 128), jnp.int32),
        pltpu.VMEM((1024, 8), jnp.float32),
        pltpu.VMEM((BINS_PER_TILE, 8), jnp.float32),
        pltpu.VMEM((BINS_PER_TILE // 128, 128), jnp.int32),
        pltpu.VMEM((256,), jnp.int32),
        pltpu.VMEM((16,), jnp.int32),
        pltpu.SemaphoreType.DMA,
        pltpu.SemaphoreType.DMA,
    ],
)
def _voxelize_sc(pts8, out8, nout, hists, offs, occ_cnts, rank_hbm,
                 ptsv, linv, ctrv, rankv, totv, rankb, idxv, zerov, nrows,
                 nidx, occv, vtmp, sem, sem2):
    tid = lax.axis_index("s")
    lane = lax.iota(jnp.int32, 16)
    zvec = jnp.zeros((16,), jnp.float32)

    def zero_fill(g, _):
        r = g * 16 + lane
        for k in range(8):
            plsc.store_scatter(zerov, [r, jnp.full((16,), k, jnp.int32)], zvec)
        return 0
    lax.fori_loop(0, 64, zero_fill, 0)

    zcps = [pltpu.async_copy(zerov,
                             out8.at[pl.ds(tid * ZROWS + j * 1024, 1024)],
                             sem)
            for j in range(31)]
    zcps.append(pltpu.async_copy(
        zerov.at[pl.ds(0, ZROWS - 31 * 1024)],
        out8.at[pl.ds(tid * ZROWS + 31 * 1024, ZROWS - 31 * 1024)], sem))
    zcps.append(pltpu.async_copy(
        zerov.at[pl.ds(0, NZROWS)], nout.at[pl.ds(tid * NZROWS, NZROWS)],
        sem))

    def zero_ctr(g, _):
        ctrv[pl.ds(g * 16, 16)] = jnp.zeros((16,), jnp.int32)
        return 0
    lax.fori_loop(0, NBINS // 16, zero_ctr, 0)

    for sub in range(2):
        pltpu.sync_copy(pts8.at[pl.ds((tid * 2 + sub) * SUB, SUB)], ptsv)

        def bin_group(g, _):
            r = g * 16 + lane
            px = plsc.load_gather(ptsv, [r, jnp.full((16,), 0, jnp.int32)])
            py = plsc.load_gather(ptsv, [r, jnp.full((16,), 1, jnp.int32)])
            pz = plsc.load_gather(ptsv, [r, jnp.full((16,), 2, jnp.int32)])
            xi = (px / jnp.float32(0.01)).astype(jnp.int32)
            yi = (py / jnp.float32(0.01)).astype(jnp.int32)
            zi = (pz / jnp.float32(1.0)).astype(jnp.int32)
            inr = ((xi >= 0) & (xi < GX) & (yi >= 0) & (yi < GY)
                   & (zi >= 0) & (zi < 1))
            ln = jnp.where(inr, zi * SENT + yi * GX + xi, SENT)
            linv[pl.ds(sub * SUB + g * 16, 16)] = ln
            old = plsc.load_gather(ctrv, [ln])
            cnt, lastm = plsc.scan_count(ln)
            plsc.store_scatter(ctrv, [ln], old + cnt, mask=lastm)
            return 0
        lax.fori_loop(0, GROUPS, bin_group, 0)

    pltpu.sync_copy(ctrv, hists.at[pl.ds(tid * NBINS, NBINS)])
    for cp in zcps:
        cp.wait()
    plsc.subcore_barrier()

    b0 = tid * BINS_PER_TILE
    hcps = [pltpu.async_copy(hists.at[pl.ds(c * NBINS + b0, BINS_PER_TILE)],
                             rankv.at[pl.ds(c * BINS_PER_TILE, BINS_PER_TILE)],
                             sem)
            for c in range(NTILES)]
    for cp in hcps:
        cp.wait()

    def prefix_group(g, _):
        acc = jnp.zeros((16,), jnp.int32)
        for c in range(NTILES):
            h = rankv[pl.ds(c * BINS_PER_TILE + g * 16, 16)]
            rankv[pl.ds(c * BINS_PER_TILE + g * 16, 16)] = acc
            acc = acc + h
        totv[pl.ds(g * 16, 16)] = acc
        return 0
    lax.fori_loop(0, BINS_PER_TILE // 16, prefix_group, 0)

    ocps = [pltpu.async_copy(rankv.at[pl.ds(c * BINS_PER_TILE, BINS_PER_TILE)],
                             offs.at[pl.ds(c * NBINS + b0, BINS_PER_TILE)],
                             sem)
            for c in range(NTILES)]

    def occ_group(g, a):
        tot = totv[pl.ds(g * 16, 16)]
        binv = b0 + g * 16 + lane
        occ = (tot > 0) & (binv < SENT)
        return a + occ.astype(jnp.int32)
    occ_acc = lax.fori_loop(0, BINS_PER_TILE // 16, occ_group,
                            jnp.zeros((16,), jnp.int32))
    vtmp[...] = jnp.full((16,), jnp.sum(occ_acc), jnp.int32)
    pltpu.sync_copy(vtmp, occ_cnts.at[pl.ds(tid * 16, 16)])
    for cp in ocps:
        cp.wait()
    plsc.subcore_barrier()

    pltpu.sync_copy(occ_cnts, occv)
    diag = plsc.load_gather(occv, [lane * 17])
    rank_base = jnp.sum(jnp.where(lane < tid, diag, 0))

    def rank_group(g, rbase):
        tot = totv[pl.ds(g * 16, 16)]
        binv = b0 + g * 16 + lane
        occ = (tot > 0) & (binv < SENT)
        occ_i = occ.astype(jnp.int32)
        excl = plsc.cumsum(occ_i) - occ_i
        rank = rbase + excl
        rankb[pl.ds(g * 16, 16)] = rank
        r = g * 16 + lane
        plsc.store_scatter(nrows, [r, jnp.full((16,), 0, jnp.int32)],
                           jnp.minimum(tot, MAXP).astype(jnp.float32))
        plsc.store_scatter(nrows, [r, jnp.full((16,), 1, jnp.int32)],
                           (binv // GX).astype(jnp.float32))
        plsc.store_scatter(nrows, [r, jnp.full((16,), 2, jnp.int32)],
                           (binv % GX).astype(jnp.float32))
        fi = g * 16 + lane
        plsc.store_scatter(nidx, [fi // 128, fi % 128],
                           jnp.where(occ, rank, MAXV + lane))
        return rbase + jnp.sum(occ_i)
    lax.fori_loop(0, BINS_PER_TILE // 16, rank_group, rank_base)

    cps = [pltpu.async_copy(nrows.at[pl.ds(j * 128, 128)],
                            nout.at[nidx.at[j]], sem)
           for j in range(BINS_PER_TILE // 128)]
    for cp in cps:
        cp.wait()
    pltpu.sync_copy(rankb, rank_hbm.at[pl.ds(b0, BINS_PER_TILE)])
    plsc.subcore_barrier()

    cpa = pltpu.async_copy(offs.at[pl.ds(tid * NBINS, NBINS)], ctrv, sem)
    cpb = pltpu.async_copy(rank_hbm, rankv, sem2)
    cpa.wait()
    cpb.wait()

    for sub in range(2):
        pltpu.sync_copy(pts8.at[pl.ds((tid * 2 + sub) * SUB, SUB)], ptsv)

        def slot_group(g, _):
            ln = linv[pl.ds(sub * SUB + g * 16, 16)]
            old = plsc.load_gather(ctrv, [ln])
            cnt, lastm = plsc.scan_count(ln)
            slot = old + cnt - 1
            plsc.store_scatter(ctrv, [ln], old + cnt, mask=lastm)
            row = plsc.load_gather(rankv, [ln])
            valid = (ln < SENT) & (slot < MAXP)
            dst = jnp.where(valid, row * MAXP + slot, ROWS + lane)
            fi = sub * SUB + g * 16 + lane
            plsc.store_scatter(idxv, [fi // 128, fi % 128], dst)
            return 0
        lax.fori_loop(0, GROUPS, slot_group, 0)

        for base in range(0, SUB // 128, 13):
            hi = min(base + 13, SUB // 128)
            scps = [pltpu.async_copy(
                        ptsv.at[pl.ds(k * 128, 128)],
                        out8.at[idxv.at[sub * (SUB // 128) + k]],
                        sem2)
                    for k in range(base, hi)]
            for cp in scps:
                cp.wait()


def kernel(points):
    n = points.shape[0]
    pts = jnp.concatenate(
        [points, jnp.full((NPAD - n, 4), 2.0, jnp.float32)], axis=0)
    pts8 = jnp.concatenate(
        [pts, jnp.zeros((NPAD, 4), jnp.float32)], axis=1)
    out8, nout, _, _, _, _ = _voxelize_sc(pts8)
    vox = out8[:ROWS, :4].reshape(MAXV, MAXP, 4)
    npts = nout[:MAXV, 0].astype(jnp.int64)
    coords = jnp.stack(
        [jnp.zeros((MAXV,), jnp.int64),
         nout[:MAXV, 1].astype(jnp.int64),
         nout[:MAXV, 2].astype(jnp.int64)], axis=1)
    return vox, coords, npts
